# Initial kernel scaffold; baseline (speedup 1.0000x reference)
import jax
import jax.numpy as jnp
from jax.experimental import pallas as pl


def kernel(input_ids, attention_mask, table):
    raise NotImplementedError("write your pallas kernel here")


# speedup vs baseline: 1.7029x; 1.7029x over previous
import functools

import jax
import jax.numpy as jnp
from jax import lax
from jax.experimental import pallas as pl
from jax.experimental.pallas import tpu as pltpu
from jax.experimental.pallas import tpu_sc as plsc

HIDDEN = 2048
HEAD_DIM = 128
ROTARY_DIM = HEAD_DIM
BASE = 10000.0

NUM_CORES = 2
NUM_SUBCORES = 16
NUM_WORKERS = NUM_CORES * NUM_SUBCORES

CHUNK = 16


def _gather_body(n_per_worker, num_chunks, ids_hbm, table_hbm, out_hbm,
                 idx_v, rows0, rows1, sem0, sem1):
    wid = lax.axis_index("s") * NUM_CORES + lax.axis_index("c")
    base = wid * n_per_worker
    pltpu.sync_copy(ids_hbm.at[pl.ds(base, n_per_worker)], idx_v)

    rows = (rows0, rows1)
    sems = (sem0, sem1)

    def start(k, buf):
        pltpu.async_copy(table_hbm.at[idx_v.at[pl.ds(k * CHUNK, CHUNK)]],
                         rows[buf], sems[buf])

    def finish(k, buf):
        pltpu.make_async_copy(table_hbm.at[idx_v.at[pl.ds(k * CHUNK, CHUNK)]],
                              rows[buf], sems[buf]).wait()
        pltpu.sync_copy(rows[buf], out_hbm.at[pl.ds(base + k * CHUNK, CHUNK)])

    start(0, 0)

    def pair(g, _):
        k0 = g * 2
        for b in range(2):
            k = k0 + b
            nxt = k + 1

            @pl.when(nxt < num_chunks)
            def _():
                start(nxt, (b + 1) % 2)

            finish(k, b)
        return ()

    lax.fori_loop(0, num_chunks // 2, pair, (), unroll=False)


def _sc_gather(ids_flat, table):
    n = ids_flat.shape[0]
    n_per_worker = n // NUM_WORKERS
    num_chunks = n_per_worker // CHUNK
    mesh = plsc.VectorSubcoreMesh(core_axis_name="c", subcore_axis_name="s",
                                  num_cores=NUM_CORES,
                                  num_subcores=NUM_SUBCORES)
    body = functools.partial(_gather_body, n_per_worker, num_chunks)
    return pl.kernel(
        body,
        out_type=jax.ShapeDtypeStruct((n, HIDDEN), table.dtype),
        mesh=mesh,
        scratch_types=[
            pltpu.VMEM((n_per_worker,), jnp.int32),
            pltpu.VMEM((CHUNK, HIDDEN), jnp.float32),
            pltpu.VMEM((CHUNK, HIDDEN), jnp.float32),
            pltpu.SemaphoreType.DMA,
            pltpu.SemaphoreType.DMA,
        ],
    )(ids_flat, table)


TBLOCK = 512


def _rope_body(invf_ref, cos_ref, sin_ref, pid_ref):
    t0 = pl.program_id(1) * TBLOCK
    t_idx = t0 + lax.broadcasted_iota(jnp.int32, (1, TBLOCK, ROTARY_DIM), 1)
    pos = t_idx.astype(jnp.float32)
    ang = pos * invf_ref[...][None, :, :]
    cos_ref[...] = jnp.cos(ang)
    sin_ref[...] = jnp.sin(ang)
    pid_ref[...] = lax.broadcasted_iota(jnp.int32, pid_ref.shape, 2)


def _tc_rope(b, t, dtype):
    inv_freq = 1.0 / (BASE ** (jnp.arange(0, ROTARY_DIM, 2,
                                          dtype=jnp.float32) / ROTARY_DIM))
    invf = jnp.concatenate([inv_freq, inv_freq])[None, :]
    grid = (b, t // TBLOCK)
    return pl.pallas_call(
        _rope_body,
        grid=grid,
        in_specs=[pl.BlockSpec((1, ROTARY_DIM), lambda i, j: (0, 0))],
        out_specs=[
            pl.BlockSpec((1, TBLOCK, ROTARY_DIM), lambda i, j: (i, j, 0)),
            pl.BlockSpec((1, TBLOCK, ROTARY_DIM), lambda i, j: (i, j, 0)),
            pl.BlockSpec((1, 1, t), lambda i, j: (i, 0, 0)),
        ],
        out_shape=[
            jax.ShapeDtypeStruct((b, t, ROTARY_DIM), dtype),
            jax.ShapeDtypeStruct((b, t, ROTARY_DIM), dtype),
            jax.ShapeDtypeStruct((b, 1, t), jnp.int32),
        ],
    )(invf)


def kernel(input_ids, attention_mask, table):
    b, t = input_ids.shape
    ids_flat = input_ids.reshape(-1)
    hidden = _sc_gather(ids_flat, table).reshape(b, t, HIDDEN)
    cos, sin, position_ids = _tc_rope(b, t, table.dtype)
    return (hidden, attention_mask, position_ids.reshape(b, t), cos, sin)


# speedup vs baseline: 1.7103x; 1.0044x over previous
import functools

import jax
import jax.numpy as jnp
from jax import lax
from jax.experimental import pallas as pl
from jax.experimental.pallas import tpu as pltpu
from jax.experimental.pallas import tpu_sc as plsc

HIDDEN = 2048
HEAD_DIM = 128
ROTARY_DIM = HEAD_DIM
BASE = 10000.0

NUM_CORES = 2
NUM_SUBCORES = 16
NUM_WORKERS = NUM_CORES * NUM_SUBCORES

CHUNK = 8
NBUF = 4


def _gather_body(n_per_worker, num_chunks, ids_hbm, table_hbm, out_hbm,
                 idx_v, rows0, rows1, rows2, rows3,
                 g0, g1, g2, g3, o0, o1, o2, o3):
    wid = lax.axis_index("s") * NUM_CORES + lax.axis_index("c")
    base = wid * n_per_worker
    pltpu.sync_copy(ids_hbm.at[pl.ds(base, n_per_worker)], idx_v)

    rows = (rows0, rows1, rows2, rows3)
    gsem = (g0, g1, g2, g3)
    osem = (o0, o1, o2, o3)

    def start(k, buf):
        pltpu.async_copy(table_hbm.at[idx_v.at[pl.ds(k * CHUNK, CHUNK)]],
                         rows[buf], gsem[buf])

    def wait_g(k, buf):
        pltpu.make_async_copy(table_hbm.at[idx_v.at[pl.ds(k * CHUNK, CHUNK)]],
                              rows[buf], gsem[buf]).wait()

    def start_out(k, buf):
        pltpu.async_copy(rows[buf], out_hbm.at[pl.ds(base + k * CHUNK, CHUNK)],
                         osem[buf])

    def wait_out(k, buf):
        pltpu.make_async_copy(rows[buf],
                              out_hbm.at[pl.ds(base + k * CHUNK, CHUNK)],
                              osem[buf]).wait()

    start(0, 0)
    start(1, 1)
    start(2, 2)
    wait_g(0, 0)
    start_out(0, 0)
    start(3, 3)

    def group(g, _):
        k0 = 1 + g * NBUF
        for b in range(NBUF):
            k = k0 + b
            buf = (1 + b) % NBUF
            wait_g(k, buf)
            start_out(k, buf)
            nxt = k + NBUF - 1
            wait_out(k - 1, (buf + NBUF - 1) % NBUF)
            start(nxt, (buf + NBUF - 1) % NBUF)
        return ()

    ngroups = (num_chunks - NBUF) // NBUF
    lax.fori_loop(0, ngroups, group, (), unroll=False)

    for k in range(num_chunks - NBUF + 1, num_chunks):
        buf = k % NBUF
        wait_g(k, buf)
        start_out(k, buf)
    for k in range(num_chunks - NBUF, num_chunks):
        wait_out(k, k % NBUF)


def _sc_gather(ids_flat, table):
    n = ids_flat.shape[0]
    n_per_worker = n // NUM_WORKERS
    num_chunks = n_per_worker // CHUNK
    mesh = plsc.VectorSubcoreMesh(core_axis_name="c", subcore_axis_name="s",
                                  num_cores=NUM_CORES,
                                  num_subcores=NUM_SUBCORES)
    body = functools.partial(_gather_body, n_per_worker, num_chunks)
    return pl.kernel(
        body,
        out_type=jax.ShapeDtypeStruct((n, HIDDEN), table.dtype),
        mesh=mesh,
        scratch_types=(
            [pltpu.VMEM((n_per_worker,), jnp.int32)]
            + [pltpu.VMEM((CHUNK, HIDDEN), jnp.float32)] * NBUF
            + [pltpu.SemaphoreType.DMA] * (2 * NBUF)
        ),
    )(ids_flat, table)


TBLOCK = 512


def _rope_body(invf_ref, cos_ref, sin_ref, pid_ref):
    t0 = pl.program_id(1) * TBLOCK
    t_idx = t0 + lax.broadcasted_iota(jnp.int32, (1, TBLOCK, ROTARY_DIM), 1)
    pos = t_idx.astype(jnp.float32)
    ang = pos * invf_ref[...][None, :, :]
    cos_ref[...] = jnp.cos(ang)
    sin_ref[...] = jnp.sin(ang)
    pid_ref[...] = lax.broadcasted_iota(jnp.int32, pid_ref.shape, 2)


def _tc_rope(b, t, dtype):
    inv_freq = 1.0 / (BASE ** (jnp.arange(0, ROTARY_DIM, 2,
                                          dtype=jnp.float32) / ROTARY_DIM))
    invf = jnp.concatenate([inv_freq, inv_freq])[None, :]
    grid = (b, t // TBLOCK)
    return pl.pallas_call(
        _rope_body,
        grid=grid,
        in_specs=[pl.BlockSpec((1, ROTARY_DIM), lambda i, j: (0, 0))],
        out_specs=[
            pl.BlockSpec((1, TBLOCK, ROTARY_DIM), lambda i, j: (i, j, 0)),
            pl.BlockSpec((1, TBLOCK, ROTARY_DIM), lambda i, j: (i, j, 0)),
            pl.BlockSpec((1, 1, t), lambda i, j: (i, 0, 0)),
        ],
        out_shape=[
            jax.ShapeDtypeStruct((b, t, ROTARY_DIM), dtype),
            jax.ShapeDtypeStruct((b, t, ROTARY_DIM), dtype),
            jax.ShapeDtypeStruct((b, 1, t), jnp.int32),
        ],
    )(invf)


def kernel(input_ids, attention_mask, table):
    b, t = input_ids.shape
    ids_flat = input_ids.reshape(-1)
    hidden = _sc_gather(ids_flat, table).reshape(b, t, HIDDEN)
    cos, sin, position_ids = _tc_rope(b, t, table.dtype)
    return (hidden, attention_mask, position_ids.reshape(b, t), cos, sin)
